# baseline (speedup 1.0000x reference)
import functools

import jax
import jax.numpy as jnp
from jax import lax
from jax.experimental import pallas as pl
from jax.experimental.pallas import tpu as pltpu
from jax.experimental.pallas import tpu_sc as plsc

BATCH = 16384
FIELDS = 26
VOCAB = 1000001

NUM_CORES = 2
NUM_SUBCORES = 16
NUM_WORKERS = NUM_CORES * NUM_SUBCORES
COLS_PER_W = BATCH // NUM_WORKERS
B_PER_W = COLS_PER_W * FIELDS

LANES = 16
NCHUNK = 2
CHUNK_COLS = COLS_PER_W // NCHUNK
CHUNK_FLAT = CHUNK_COLS * FIELDS
VECS_PER_ROW = CHUNK_COLS // LANES

STAGE_CHUNK = 62592
STAGE_TAIL = 61056
STAGE_REM_OFF = 15 * STAGE_CHUNK + STAGE_TAIL
STAGE_REM = VOCAB - STAGE_REM_OFF

_mesh = plsc.VectorSubcoreMesh(core_axis_name="c", subcore_axis_name="s")


@functools.partial(
    pl.kernel,
    mesh=_mesh,
    out_type=jax.ShapeDtypeStruct((FIELDS, BATCH), jnp.float32),
    scratch_types=[
        pltpu.VMEM_SHARED((VOCAB,), jnp.float32),
        pltpu.VMEM((FIELDS, COLS_PER_W), jnp.int32),
        pltpu.VMEM((B_PER_W,), jnp.int32),
        pltpu.VMEM((B_PER_W,), jnp.float32),
        pltpu.VMEM((FIELDS, COLS_PER_W), jnp.float32),
        pltpu.VMEM((128,), jnp.float32),
        pltpu.SemaphoreType.DMA,
        pltpu.SemaphoreType.DMA,
        pltpu.SemaphoreType.DMA,
        pltpu.SemaphoreType.DMA,
    ],
    compiler_params=pltpu.CompilerParams(
        use_tc_tiling_on_sc=True, needs_layout_passes=False
    ),
)
def _gather_kernel(
    w_hbm, xt_hbm, out_hbm,
    table_s, xin_v, idx_v, vals_v, xout_v, tail_v,
    sem_in, sem_g, sem_out, sem_t,
):
    cid = lax.axis_index("c")
    sid = lax.axis_index("s")
    wid = sid * NUM_CORES + cid
    col0 = wid * COLS_PER_W

    in_copies = [
        pltpu.async_copy(
            xt_hbm.at[:, pl.ds(col0 + q * CHUNK_COLS, CHUNK_COLS)],
            xin_v.at[:, pl.ds(q * CHUNK_COLS, CHUNK_COLS)],
            sem_in,
        )
        for q in range(NCHUNK)
    ]

    @pl.when(sid < NUM_SUBCORES - 1)
    def _():
        off = pl.multiple_of(sid * STAGE_CHUNK, 128)
        pltpu.async_copy(
            w_hbm.at[pl.ds(off, STAGE_CHUNK)], table_s.at[pl.ds(off, STAGE_CHUNK)],
            sem_t,
        )

    @pl.when(sid == NUM_SUBCORES - 1)
    def _():
        off = (NUM_SUBCORES - 1) * STAGE_CHUNK
        pltpu.async_copy(
            w_hbm.at[pl.ds(off, STAGE_TAIL)], table_s.at[pl.ds(off, STAGE_TAIL)],
            sem_t,
        )
        pltpu.sync_copy(w_hbm.at[pl.ds(STAGE_REM_OFF, STAGE_REM)], tail_v.at[pl.ds(0, STAGE_REM)])
        pltpu.sync_copy(tail_v.at[pl.ds(0, STAGE_REM)], table_s.at[pl.ds(STAGE_REM_OFF, STAGE_REM)])

    for q in range(NCHUNK):
        in_copies[q].wait()
        cbase = q * CHUNK_COLS
        fbase = q * CHUNK_FLAT

        def compact_row(f, carry, cbase=cbase, fbase=fbase):
            base = fbase + f * CHUNK_COLS
            for v in range(VECS_PER_ROW):
                idx_v[pl.ds(base + v * LANES, LANES)] = xin_v[
                    f, pl.ds(cbase + v * LANES, LANES)
                ]
            return carry

        lax.fori_loop(0, FIELDS, compact_row, 0)

    @pl.when(sid < NUM_SUBCORES - 1)
    def _():
        pltpu.make_async_copy(
            w_hbm.at[pl.ds(0, STAGE_CHUNK)], table_s.at[pl.ds(0, STAGE_CHUNK)], sem_t
        ).wait()

    @pl.when(sid == NUM_SUBCORES - 1)
    def _():
        pltpu.make_async_copy(
            w_hbm.at[pl.ds(0, STAGE_TAIL)], table_s.at[pl.ds(0, STAGE_TAIL)], sem_t
        ).wait()

    plsc.subcore_barrier()

    gathers = []
    for q in range(NCHUNK):
        fbase = q * CHUNK_FLAT
        gathers.append(
            pltpu.async_copy(
                table_s.at[idx_v.at[pl.ds(fbase, CHUNK_FLAT)]],
                vals_v.at[pl.ds(fbase, CHUNK_FLAT)],
                sem_g,
            )
        )

    out_copies = []
    for q in range(NCHUNK):
        gathers[q].wait()
        cbase = q * CHUNK_COLS
        fbase = q * CHUNK_FLAT

        def expand_row(f, carry, cbase=cbase, fbase=fbase):
            base = fbase + f * CHUNK_COLS
            for v in range(VECS_PER_ROW):
                xout_v[f, pl.ds(cbase + v * LANES, LANES)] = vals_v[
                    pl.ds(base + v * LANES, LANES)
                ]
            return carry

        lax.fori_loop(0, FIELDS, expand_row, 0)
        out_copies.append(
            pltpu.async_copy(
                xout_v.at[:, pl.ds(cbase, CHUNK_COLS)],
                out_hbm.at[:, pl.ds(col0 + cbase, CHUNK_COLS)],
                sem_out,
            )
        )

    for c in out_copies:
        c.wait()


def kernel(x, weights):
    out_t = _gather_kernel(weights, x.astype(jnp.int32).T)
    return out_t.T


# speedup vs baseline: 1.0142x; 1.0142x over previous
import functools

import jax
import jax.numpy as jnp
from jax import lax
from jax.experimental import pallas as pl
from jax.experimental.pallas import tpu as pltpu
from jax.experimental.pallas import tpu_sc as plsc

BATCH = 16384
FIELDS = 26
VOCAB = 1000001

NUM_CORES = 2
NUM_SUBCORES = 16
NUM_WORKERS = NUM_CORES * NUM_SUBCORES
COLS_PER_W = BATCH // NUM_WORKERS
B_PER_W = COLS_PER_W * FIELDS

LANES = 16
NCHUNK = 4
CHUNK_COLS = COLS_PER_W // NCHUNK
CHUNK_FLAT = CHUNK_COLS * FIELDS
VECS_PER_ROW = CHUNK_COLS // LANES

STAGE_CHUNK = 62592
STAGE_TAIL = 61056
STAGE_REM_OFF = 15 * STAGE_CHUNK + STAGE_TAIL
STAGE_REM = VOCAB - STAGE_REM_OFF

_mesh = plsc.VectorSubcoreMesh(core_axis_name="c", subcore_axis_name="s")


@functools.partial(
    pl.kernel,
    mesh=_mesh,
    out_type=jax.ShapeDtypeStruct((FIELDS, BATCH), jnp.float32),
    scratch_types=[
        pltpu.VMEM_SHARED((VOCAB,), jnp.float32),
        pltpu.VMEM((FIELDS, COLS_PER_W), jnp.int32),
        pltpu.VMEM((B_PER_W,), jnp.int32),
        pltpu.VMEM((B_PER_W,), jnp.float32),
        pltpu.VMEM((FIELDS, COLS_PER_W), jnp.float32),
        pltpu.VMEM((128,), jnp.float32),
        pltpu.SemaphoreType.DMA,
        pltpu.SemaphoreType.DMA,
        pltpu.SemaphoreType.DMA,
        pltpu.SemaphoreType.DMA,
    ],
    compiler_params=pltpu.CompilerParams(
        use_tc_tiling_on_sc=True, needs_layout_passes=False
    ),
)
def _gather_kernel(
    w_hbm, xt_hbm, out_hbm,
    table_s, xin_v, idx_v, vals_v, xout_v, tail_v,
    sem_in, sem_g, sem_out, sem_t,
):
    cid = lax.axis_index("c")
    sid = lax.axis_index("s")
    wid = sid * NUM_CORES + cid
    col0 = wid * COLS_PER_W

    in_copies = [
        pltpu.async_copy(
            xt_hbm.at[:, pl.ds(col0 + q * CHUNK_COLS, CHUNK_COLS)],
            xin_v.at[:, pl.ds(q * CHUNK_COLS, CHUNK_COLS)],
            sem_in,
        )
        for q in range(NCHUNK)
    ]

    @pl.when(sid < NUM_SUBCORES - 1)
    def _():
        off = pl.multiple_of(sid * STAGE_CHUNK, 128)
        pltpu.async_copy(
            w_hbm.at[pl.ds(off, STAGE_CHUNK)], table_s.at[pl.ds(off, STAGE_CHUNK)],
            sem_t,
        )

    @pl.when(sid == NUM_SUBCORES - 1)
    def _():
        off = (NUM_SUBCORES - 1) * STAGE_CHUNK
        pltpu.async_copy(
            w_hbm.at[pl.ds(off, STAGE_TAIL)], table_s.at[pl.ds(off, STAGE_TAIL)],
            sem_t,
        )
        pltpu.sync_copy(w_hbm.at[pl.ds(STAGE_REM_OFF, STAGE_REM)], tail_v.at[pl.ds(0, STAGE_REM)])
        pltpu.sync_copy(tail_v.at[pl.ds(0, STAGE_REM)], table_s.at[pl.ds(STAGE_REM_OFF, STAGE_REM)])

    for q in range(NCHUNK):
        in_copies[q].wait()
        cbase = q * CHUNK_COLS
        fbase = q * CHUNK_FLAT

        def compact_row(f, carry, cbase=cbase, fbase=fbase):
            base = fbase + f * CHUNK_COLS
            for v in range(VECS_PER_ROW):
                idx_v[pl.ds(base + v * LANES, LANES)] = xin_v[
                    f, pl.ds(cbase + v * LANES, LANES)
                ]
            return carry

        lax.fori_loop(0, FIELDS, compact_row, 0)

    @pl.when(sid < NUM_SUBCORES - 1)
    def _():
        pltpu.make_async_copy(
            w_hbm.at[pl.ds(0, STAGE_CHUNK)], table_s.at[pl.ds(0, STAGE_CHUNK)], sem_t
        ).wait()

    @pl.when(sid == NUM_SUBCORES - 1)
    def _():
        pltpu.make_async_copy(
            w_hbm.at[pl.ds(0, STAGE_TAIL)], table_s.at[pl.ds(0, STAGE_TAIL)], sem_t
        ).wait()

    plsc.subcore_barrier()

    gathers = []
    for q in range(NCHUNK):
        fbase = q * CHUNK_FLAT
        gathers.append(
            pltpu.async_copy(
                table_s.at[idx_v.at[pl.ds(fbase, CHUNK_FLAT)]],
                vals_v.at[pl.ds(fbase, CHUNK_FLAT)],
                sem_g,
            )
        )

    out_copies = []
    for q in range(NCHUNK):
        gathers[q].wait()
        cbase = q * CHUNK_COLS
        fbase = q * CHUNK_FLAT

        def expand_row(f, carry, cbase=cbase, fbase=fbase):
            base = fbase + f * CHUNK_COLS
            for v in range(VECS_PER_ROW):
                xout_v[f, pl.ds(cbase + v * LANES, LANES)] = vals_v[
                    pl.ds(base + v * LANES, LANES)
                ]
            return carry

        lax.fori_loop(0, FIELDS, expand_row, 0)
        out_copies.append(
            pltpu.async_copy(
                xout_v.at[:, pl.ds(cbase, CHUNK_COLS)],
                out_hbm.at[:, pl.ds(col0 + cbase, CHUNK_COLS)],
                sem_out,
            )
        )

    for c in out_copies:
        c.wait()


def kernel(x, weights):
    out_t = _gather_kernel(weights, x.astype(jnp.int32).T)
    return out_t.T
